# baseline (speedup 1.0000x reference)
import functools
import jax
import jax.numpy as jnp
from jax import lax
from jax.experimental import pallas as pl
from jax.experimental.pallas import tpu as pltpu
from jax.experimental.pallas import tpu_sc as plsc

N = 10000
D = 128
E = 320000

NC = 2
NS = 16
NW = NC * NS
L = 16

NP = 10240
CH = 128
NCH = 80
G = 16
NG = NCH // G
EPT = CH * NCH
EP = EPT * NW
SEG = NP // NS

CH2 = 128
TCH = EP // CH2
C0 = 128
C1 = (TCH // NS) - C0

_mesh = plsc.VectorSubcoreMesh(core_axis_name="c", subcore_axis_name="s",
                               num_cores=NC, num_subcores=NS)
_sc_params = pltpu.CompilerParams(needs_layout_passes=False)


def _zero16():
    return jnp.zeros((L,), jnp.float32)


def _newton_rsqrt(x):
    xm = jnp.maximum(x, 1e-12)
    i = plsc.bitcast(xm, jnp.int32)
    i = jnp.int32(0x5F3759DF) - lax.shift_right_logical(i, 1)
    y = plsc.bitcast(i, jnp.float32)
    for _ in range(3):
        y = y * (1.5 - 0.5 * xm * y * y)
    return jnp.where(x > 0.0, y, 0.0)


@functools.partial(
    pl.kernel,
    out_type=jax.ShapeDtypeStruct((NW, NCH, CH), jnp.float32),
    mesh=_mesh,
    scratch_types=[
        pltpu.VMEM((G, CH), jnp.int32),
        pltpu.VMEM((G, CH), jnp.float32),
        pltpu.VMEM((SEG,), jnp.float32),
        pltpu.VMEM((NP,), jnp.float32),
        pltpu.VMEM((G, CH), jnp.int32),
        pltpu.VMEM((G, CH), jnp.float32),
        pltpu.VMEM_SHARED((NP,), jnp.float32),
    ],
    compiler_params=_sc_params,
)
def _norm_kernel(src_hbm, dst_hbm, w_hbm, norm_out,
                 dstg, wg, degv, dinv_v, srcg, normg, deg_sh):
    cid = lax.axis_index("c")
    sid = lax.axis_index("s")
    wid = cid * NS + sid

    for j in range(SEG // L):
        degv[pl.ds(j * L, L)] = _zero16()
    pltpu.sync_copy(degv, deg_sh.at[pl.ds(sid * SEG, SEG)])
    plsc.subcore_barrier()

    def deg_group(arg, _):
        wblk, g = arg // NG, arg % NG
        pltpu.sync_copy(dst_hbm.at[wblk * NS + sid, pl.ds(g * G, G), :], dstg)
        pltpu.sync_copy(w_hbm.at[wblk * NS + sid, pl.ds(g * G, G), :], wg)

        def deg_chunk(k, _):
            pltpu.sync_copy(wg.at[k], deg_sh.at[dstg.at[k]], add=True)
            return 0
        lax.fori_loop(0, G, deg_chunk, 0)
        return 0
    lax.fori_loop(0, NC * NG, deg_group, 0)
    plsc.subcore_barrier()

    pltpu.sync_copy(deg_sh.at[pl.ds(sid * SEG, SEG)], degv)
    def dinv_body(j, _):
        degv[pl.ds(j * L, L)] = _newton_rsqrt(degv[pl.ds(j * L, L)])
        return 0
    lax.fori_loop(0, SEG // L, dinv_body, 0)
    pltpu.sync_copy(degv, deg_sh.at[pl.ds(sid * SEG, SEG)])
    plsc.subcore_barrier()

    pltpu.sync_copy(deg_sh, dinv_v)
    def norm_group(g, _):
        pltpu.sync_copy(src_hbm.at[wid, pl.ds(g * G, G), :], srcg)
        pltpu.sync_copy(dst_hbm.at[wid, pl.ds(g * G, G), :], dstg)
        pltpu.sync_copy(w_hbm.at[wid, pl.ds(g * G, G), :], wg)

        def norm_chunk(k, _):
            for j in range(CH // L):
                sv = srcg[k, pl.ds(j * L, L)]
                dv = dstg[k, pl.ds(j * L, L)]
                we = wg[k, pl.ds(j * L, L)]
                normg[k, pl.ds(j * L, L)] = (
                    plsc.load_gather(dinv_v, [sv]) * we *
                    plsc.load_gather(dinv_v, [dv]))
            return 0
        lax.fori_loop(0, G, norm_chunk, 0)
        pltpu.sync_copy(normg, norm_out.at[wid, pl.ds(g * G, G), :])
        return 0
    lax.fori_loop(0, NG, norm_group, 0)


@functools.partial(
    pl.kernel,
    out_type=jax.ShapeDtypeStruct((NC, NP, D), jnp.float32),
    mesh=_mesh,
    scratch_types=[
        pltpu.VMEM((2, G, CH2), jnp.int32),
        pltpu.VMEM((2, G, CH2), jnp.int32),
        pltpu.VMEM((2, G, CH2), jnp.float32),
        pltpu.VMEM((2, CH2, D), jnp.float32),
        pltpu.VMEM_SHARED((NP, D), jnp.float32),
        pltpu.SemaphoreType.DMA,
        pltpu.SemaphoreType.DMA,
        pltpu.SemaphoreType.DMA,
        pltpu.SemaphoreType.DMA,
        pltpu.SemaphoreType.DMA,
    ],
    compiler_params=_sc_params,
)
def _scatter_kernel(src_hbm, dst_hbm, norm_hbm, x_hbm, p_out,
                    srcg, dstg, normg, rows, p_sh,
                    gsem0, gsem1, ssem0, ssem1, stsem):
    cid = lax.axis_index("c")
    sid = lax.axis_index("s")
    gsem = (gsem0, gsem1)
    ssem = (ssem0, ssem1)

    nch = jnp.where(cid == 0, C0, C1)
    ng = nch // G
    base = jnp.where(cid == 0, sid * C0, NS * C0 + sid * C1)

    def gbk(c):
        return (c // G) % 2, c % G

    def stage_group(g, gb):
        pltpu.async_copy(src_hbm.at[pl.ds(base + g * G, G), :],
                         srcg.at[gb], stsem)
        pltpu.async_copy(dst_hbm.at[pl.ds(base + g * G, G), :],
                         dstg.at[gb], stsem)
        pltpu.async_copy(norm_hbm.at[pl.ds(base + g * G, G), :],
                         normg.at[gb], stsem)

    def wait_stage(gb):
        pltpu.make_async_copy(src_hbm.at[pl.ds(0, G), :],
                              srcg.at[gb], stsem).wait()
        pltpu.make_async_copy(dst_hbm.at[pl.ds(0, G), :],
                              dstg.at[gb], stsem).wait()
        pltpu.make_async_copy(norm_hbm.at[pl.ds(0, G), :],
                              normg.at[gb], stsem).wait()

    def issue_gather(c, b):
        gb, k = gbk(c)
        pltpu.async_copy(x_hbm.at[srcg.at[gb, k]], rows.at[b], gsem[b])

    def wait_gather(c, b):
        gb, k = gbk(c)
        pltpu.make_async_copy(x_hbm.at[srcg.at[gb, k]], rows.at[b],
                              gsem[b]).wait()

    def issue_scatter(c, b):
        gb, k = gbk(c)
        pltpu.async_copy(rows.at[b], p_sh.at[dstg.at[gb, k]], ssem[b],
                         add=True)

    def wait_scatter(c, b):
        gb, k = gbk(c)
        pltpu.make_async_copy(rows.at[b], p_sh.at[dstg.at[gb, k]],
                              ssem[b]).wait()

    def scale(c, b):
        gb, k = gbk(c)

        def ebody(i, _):
            for e in (2 * i, 2 * i + 1):
                nrep = plsc.load_gather(
                    normg, [jnp.full((L,), gb, jnp.int32),
                            jnp.full((L,), k, jnp.int32),
                            jnp.full((L,), e, jnp.int32)])
                for j in range(D // L):
                    rows[b, e, pl.ds(j * L, L)] = (
                        rows[b, e, pl.ds(j * L, L)] * nrep)
            return 0
        lax.fori_loop(0, CH2 // 2, ebody, 0)

    def zbody(i, _):
        for j in range(D // L):
            rows[0, i, pl.ds(j * L, L)] = _zero16()
        return 0
    lax.fori_loop(0, CH2, zbody, 0)
    for k in range(SEG // CH2):
        pltpu.sync_copy(rows.at[0],
                        p_sh.at[pl.ds(sid * SEG + k * CH2, CH2), :])

    pltpu.sync_copy(src_hbm.at[pl.ds(base, G), :], srcg.at[0])
    pltpu.sync_copy(dst_hbm.at[pl.ds(base, G), :], dstg.at[0])
    pltpu.sync_copy(norm_hbm.at[pl.ds(base, G), :], normg.at[0])
    issue_gather(0, 0)
    plsc.subcore_barrier()

    def pair(t, _):
        for q in range(2):
            i = 2 * t + q

            wait_gather(i, q)

            @pl.when(i > 0)
            def _():
                wait_scatter(i - 1, 1 - q)

            @pl.when((i % G == 0) & (i // G + 1 < ng))
            def _():
                stage_group(i // G + 1, (i // G + 1) % 2)

            @pl.when(i + 1 < nch)
            def _():
                @pl.when((i + 1) % G == 0)
                def _():
                    wait_stage(((i + 1) // G) % 2)
                issue_gather(i + 1, 1 - q)

            scale(i, q)
            issue_scatter(i, q)
        return 0
    lax.fori_loop(0, nch // 2, pair, 0)
    wait_scatter(nch - 1, 1)
    plsc.subcore_barrier()
    pltpu.sync_copy(p_sh.at[pl.ds(sid * SEG, SEG), :],
                    p_out.at[cid, pl.ds(sid * SEG, SEG), :])


def _gates_body(x_ref, p_ref, wxz_ref, vxz_ref, wxh_ref, vxh_ref,
                bxz_ref, bhz_ref, bxh_ref, bhh_ref, out_ref):
    x = x_ref[...]
    p = p_ref[0] + p_ref[1]
    az = (jnp.dot(x, vxz_ref[...], preferred_element_type=jnp.float32)
          - jnp.dot(p, wxz_ref[...], preferred_element_type=jnp.float32)
          + bxz_ref[...])
    ah = (jnp.dot(x, vxh_ref[...], preferred_element_type=jnp.float32)
          - jnp.dot(p, wxh_ref[...], preferred_element_type=jnp.float32)
          + bxh_ref[...])
    z = jax.nn.sigmoid(jax.nn.relu(az) + jax.nn.relu(bhz_ref[...]))
    ht = jnp.tanh(jax.nn.relu(ah) + jax.nn.relu(bhh_ref[...]))
    out_ref[...] = z * ht


_RB = 2000


def _gates(X, p_part, Wxz, Vxz, Wxh, Vxh, bxz, bhz, bxh, bhh):
    wspec = pl.BlockSpec((D, D), lambda i: (0, 0))
    bspec = pl.BlockSpec((1, D), lambda i: (0, 0))
    return pl.pallas_call(
        _gates_body,
        grid=(N // _RB,),
        in_specs=[
            pl.BlockSpec((_RB, D), lambda i: (i, 0)),
            pl.BlockSpec((NC, _RB, D), lambda i: (0, i, 0)),
            wspec, wspec, wspec, wspec,
            bspec, bspec, bspec, bspec,
        ],
        out_specs=pl.BlockSpec((_RB, D), lambda i: (i, 0)),
        out_shape=jax.ShapeDtypeStruct((N, D), jnp.float32),
    )(X, p_part, Wxz, Vxz, Wxh, Vxh,
      bxz.reshape(1, D), bhz.reshape(1, D),
      bxh.reshape(1, D), bhh.reshape(1, D))


def kernel(X, edge_index, edge_weight, Wxz, Vxz, bxz, Whz, Vhz, bhz,
           Wxr, Vxr, bxr, Whr, Vhr, bhr, Wxh, Vxh, bxh, Whh, Vhh, bhh):
    pad = EP - E
    src = jnp.concatenate([edge_index[0], jnp.zeros((pad,), jnp.int32)])
    dst = jnp.concatenate([edge_index[1], jnp.zeros((pad,), jnp.int32)])
    w = jnp.concatenate([edge_weight, jnp.zeros((pad,), jnp.float32)])
    src3 = src.reshape(NW, NCH, CH)
    dst3 = dst.reshape(NW, NCH, CH)
    w3 = w.reshape(NW, NCH, CH)

    norm3 = _norm_kernel(src3, dst3, w3)
    p_part = _scatter_kernel(src.reshape(TCH, CH2), dst.reshape(TCH, CH2),
                             norm3.reshape(TCH, CH2), X)
    return _gates(X, p_part, Wxz, Vxz, Wxh, Vxh, bxz, bhz, bxh, bhh)


# speedup vs baseline: 1.1074x; 1.1074x over previous
import functools
import jax
import jax.numpy as jnp
from jax import lax
from jax.experimental import pallas as pl
from jax.experimental.pallas import tpu as pltpu
from jax.experimental.pallas import tpu_sc as plsc

N = 10000
D = 128
E = 320000

NC = 2
NS = 16
NW = NC * NS
L = 16

NP = 10240
CH = 128
NCH = 80
G = 16
NG = NCH // G
EPT = CH * NCH
EP = EPT * NW
SEG = NP // NS

CH2 = 128
TCH = EP // CH2
C0 = 144
C1 = (TCH // NS) - C0

_mesh = plsc.VectorSubcoreMesh(core_axis_name="c", subcore_axis_name="s",
                               num_cores=NC, num_subcores=NS)
_sc_params = pltpu.CompilerParams(needs_layout_passes=False)


def _zero16():
    return jnp.zeros((L,), jnp.float32)


def _newton_rsqrt(x):
    xm = jnp.maximum(x, 1e-12)
    i = plsc.bitcast(xm, jnp.int32)
    i = jnp.int32(0x5F3759DF) - lax.shift_right_logical(i, 1)
    y = plsc.bitcast(i, jnp.float32)
    for _ in range(3):
        y = y * (1.5 - 0.5 * xm * y * y)
    return jnp.where(x > 0.0, y, 0.0)


@functools.partial(
    pl.kernel,
    out_type=jax.ShapeDtypeStruct((NW, NCH, CH), jnp.float32),
    mesh=_mesh,
    scratch_types=[
        pltpu.VMEM((G, CH), jnp.int32),
        pltpu.VMEM((G, CH), jnp.float32),
        pltpu.VMEM((SEG,), jnp.float32),
        pltpu.VMEM((NP,), jnp.float32),
        pltpu.VMEM((G, CH), jnp.int32),
        pltpu.VMEM((G, CH), jnp.float32),
        pltpu.VMEM_SHARED((NP,), jnp.float32),
    ],
    compiler_params=_sc_params,
)
def _norm_kernel(src_hbm, dst_hbm, w_hbm, norm_out,
                 dstg, wg, degv, dinv_v, srcg, normg, deg_sh):
    cid = lax.axis_index("c")
    sid = lax.axis_index("s")
    wid = cid * NS + sid

    for j in range(SEG // L):
        degv[pl.ds(j * L, L)] = _zero16()
    pltpu.sync_copy(degv, deg_sh.at[pl.ds(sid * SEG, SEG)])
    plsc.subcore_barrier()

    def deg_group(arg, _):
        wblk, g = arg // NG, arg % NG
        pltpu.sync_copy(dst_hbm.at[wblk * NS + sid, pl.ds(g * G, G), :], dstg)
        pltpu.sync_copy(w_hbm.at[wblk * NS + sid, pl.ds(g * G, G), :], wg)

        def deg_chunk(k, _):
            pltpu.sync_copy(wg.at[k], deg_sh.at[dstg.at[k]], add=True)
            return 0
        lax.fori_loop(0, G, deg_chunk, 0)
        return 0
    lax.fori_loop(0, NC * NG, deg_group, 0)
    plsc.subcore_barrier()

    pltpu.sync_copy(deg_sh.at[pl.ds(sid * SEG, SEG)], degv)
    def dinv_body(j, _):
        degv[pl.ds(j * L, L)] = _newton_rsqrt(degv[pl.ds(j * L, L)])
        return 0
    lax.fori_loop(0, SEG // L, dinv_body, 0)
    pltpu.sync_copy(degv, deg_sh.at[pl.ds(sid * SEG, SEG)])
    plsc.subcore_barrier()

    pltpu.sync_copy(deg_sh, dinv_v)
    def norm_group(g, _):
        pltpu.sync_copy(src_hbm.at[wid, pl.ds(g * G, G), :], srcg)
        pltpu.sync_copy(dst_hbm.at[wid, pl.ds(g * G, G), :], dstg)
        pltpu.sync_copy(w_hbm.at[wid, pl.ds(g * G, G), :], wg)

        def norm_chunk(k, _):
            for j in range(CH // L):
                sv = srcg[k, pl.ds(j * L, L)]
                dv = dstg[k, pl.ds(j * L, L)]
                we = wg[k, pl.ds(j * L, L)]
                normg[k, pl.ds(j * L, L)] = (
                    plsc.load_gather(dinv_v, [sv]) * we *
                    plsc.load_gather(dinv_v, [dv]))
            return 0
        lax.fori_loop(0, G, norm_chunk, 0)
        pltpu.sync_copy(normg, norm_out.at[wid, pl.ds(g * G, G), :])
        return 0
    lax.fori_loop(0, NG, norm_group, 0)


@functools.partial(
    pl.kernel,
    out_type=jax.ShapeDtypeStruct((NC, NP, D), jnp.float32),
    mesh=_mesh,
    scratch_types=[
        pltpu.VMEM((2, G, CH2), jnp.int32),
        pltpu.VMEM((2, G, CH2), jnp.int32),
        pltpu.VMEM((2, G, CH2), jnp.float32),
        pltpu.VMEM((2, CH2, D), jnp.float32),
        pltpu.VMEM_SHARED((NP, D), jnp.float32),
        pltpu.SemaphoreType.DMA,
        pltpu.SemaphoreType.DMA,
        pltpu.SemaphoreType.DMA,
        pltpu.SemaphoreType.DMA,
        pltpu.SemaphoreType.DMA,
    ],
    compiler_params=_sc_params,
)
def _scatter_kernel(src_hbm, dst_hbm, norm_hbm, x_hbm, p_out,
                    srcg, dstg, normg, rows, p_sh,
                    gsem0, gsem1, ssem0, ssem1, stsem):
    cid = lax.axis_index("c")
    sid = lax.axis_index("s")
    gsem = (gsem0, gsem1)
    ssem = (ssem0, ssem1)

    nch = jnp.where(cid == 0, C0, C1)
    ng = nch // G
    base = jnp.where(cid == 0, sid * C0, NS * C0 + sid * C1)

    def gbk(c):
        return (c // G) % 2, c % G

    def stage_group(g, gb):
        pltpu.async_copy(src_hbm.at[pl.ds(base + g * G, G), :],
                         srcg.at[gb], stsem)
        pltpu.async_copy(dst_hbm.at[pl.ds(base + g * G, G), :],
                         dstg.at[gb], stsem)
        pltpu.async_copy(norm_hbm.at[pl.ds(base + g * G, G), :],
                         normg.at[gb], stsem)

    def wait_stage(gb):
        pltpu.make_async_copy(src_hbm.at[pl.ds(0, G), :],
                              srcg.at[gb], stsem).wait()
        pltpu.make_async_copy(dst_hbm.at[pl.ds(0, G), :],
                              dstg.at[gb], stsem).wait()
        pltpu.make_async_copy(norm_hbm.at[pl.ds(0, G), :],
                              normg.at[gb], stsem).wait()

    def issue_gather(c, b):
        gb, k = gbk(c)
        pltpu.async_copy(x_hbm.at[srcg.at[gb, k]], rows.at[b], gsem[b])

    def wait_gather(c, b):
        gb, k = gbk(c)
        pltpu.make_async_copy(x_hbm.at[srcg.at[gb, k]], rows.at[b],
                              gsem[b]).wait()

    def issue_scatter(c, b):
        gb, k = gbk(c)
        pltpu.async_copy(rows.at[b], p_sh.at[dstg.at[gb, k]], ssem[b],
                         add=True)

    def wait_scatter(c, b):
        gb, k = gbk(c)
        pltpu.make_async_copy(rows.at[b], p_sh.at[dstg.at[gb, k]],
                              ssem[b]).wait()

    def scale(c, b):
        gb, k = gbk(c)

        def ebody(i, _):
            for e in (2 * i, 2 * i + 1):
                nrep = plsc.load_gather(
                    normg, [jnp.full((L,), gb, jnp.int32),
                            jnp.full((L,), k, jnp.int32),
                            jnp.full((L,), e, jnp.int32)])
                for j in range(D // L):
                    rows[b, e, pl.ds(j * L, L)] = (
                        rows[b, e, pl.ds(j * L, L)] * nrep)
            return 0
        lax.fori_loop(0, CH2 // 2, ebody, 0)

    def zbody(i, _):
        for j in range(D // L):
            rows[0, i, pl.ds(j * L, L)] = _zero16()
        return 0
    lax.fori_loop(0, CH2, zbody, 0)
    for k in range(SEG // CH2):
        pltpu.sync_copy(rows.at[0],
                        p_sh.at[pl.ds(sid * SEG + k * CH2, CH2), :])

    pltpu.sync_copy(src_hbm.at[pl.ds(base, G), :], srcg.at[0])
    pltpu.sync_copy(dst_hbm.at[pl.ds(base, G), :], dstg.at[0])
    pltpu.sync_copy(norm_hbm.at[pl.ds(base, G), :], normg.at[0])
    issue_gather(0, 0)
    plsc.subcore_barrier()

    def pair(t, _):
        for q in range(2):
            i = 2 * t + q

            wait_gather(i, q)

            @pl.when(i > 0)
            def _():
                wait_scatter(i - 1, 1 - q)

            @pl.when((i % G == 0) & (i // G + 1 < ng))
            def _():
                stage_group(i // G + 1, (i // G + 1) % 2)

            @pl.when(i + 1 < nch)
            def _():
                @pl.when((i + 1) % G == 0)
                def _():
                    wait_stage(((i + 1) // G) % 2)
                issue_gather(i + 1, 1 - q)

            scale(i, q)
            issue_scatter(i, q)
        return 0
    lax.fori_loop(0, nch // 2, pair, 0)
    wait_scatter(nch - 1, 1)
    plsc.subcore_barrier()
    pltpu.sync_copy(p_sh.at[pl.ds(sid * SEG, SEG), :],
                    p_out.at[cid, pl.ds(sid * SEG, SEG), :])


def _gates_body(x_ref, p_ref, wxz_ref, vxz_ref, wxh_ref, vxh_ref,
                bxz_ref, bhz_ref, bxh_ref, bhh_ref, out_ref):
    x = x_ref[...]
    p = p_ref[0] + p_ref[1]
    az = (jnp.dot(x, vxz_ref[...], preferred_element_type=jnp.float32)
          - jnp.dot(p, wxz_ref[...], preferred_element_type=jnp.float32)
          + bxz_ref[...])
    ah = (jnp.dot(x, vxh_ref[...], preferred_element_type=jnp.float32)
          - jnp.dot(p, wxh_ref[...], preferred_element_type=jnp.float32)
          + bxh_ref[...])
    z = jax.nn.sigmoid(jax.nn.relu(az) + jax.nn.relu(bhz_ref[...]))
    ht = jnp.tanh(jax.nn.relu(ah) + jax.nn.relu(bhh_ref[...]))
    out_ref[...] = z * ht


_RB = 2000


def _gates(X, p_part, Wxz, Vxz, Wxh, Vxh, bxz, bhz, bxh, bhh):
    wspec = pl.BlockSpec((D, D), lambda i: (0, 0))
    bspec = pl.BlockSpec((1, D), lambda i: (0, 0))
    return pl.pallas_call(
        _gates_body,
        grid=(N // _RB,),
        in_specs=[
            pl.BlockSpec((_RB, D), lambda i: (i, 0)),
            pl.BlockSpec((NC, _RB, D), lambda i: (0, i, 0)),
            wspec, wspec, wspec, wspec,
            bspec, bspec, bspec, bspec,
        ],
        out_specs=pl.BlockSpec((_RB, D), lambda i: (i, 0)),
        out_shape=jax.ShapeDtypeStruct((N, D), jnp.float32),
    )(X, p_part, Wxz, Vxz, Wxh, Vxh,
      bxz.reshape(1, D), bhz.reshape(1, D),
      bxh.reshape(1, D), bhh.reshape(1, D))


def kernel(X, edge_index, edge_weight, Wxz, Vxz, bxz, Whz, Vhz, bhz,
           Wxr, Vxr, bxr, Whr, Vhr, bhr, Wxh, Vxh, bxh, Whh, Vhh, bhh):
    pad = EP - E
    src = jnp.concatenate([edge_index[0], jnp.zeros((pad,), jnp.int32)])
    dst = jnp.concatenate([edge_index[1], jnp.zeros((pad,), jnp.int32)])
    w = jnp.concatenate([edge_weight, jnp.zeros((pad,), jnp.float32)])
    src3 = src.reshape(NW, NCH, CH)
    dst3 = dst.reshape(NW, NCH, CH)
    w3 = w.reshape(NW, NCH, CH)

    norm3 = _norm_kernel(src3, dst3, w3)
    p_part = _scatter_kernel(src.reshape(TCH, CH2), dst.reshape(TCH, CH2),
                             norm3.reshape(TCH, CH2), X)
    return _gates(X, p_part, Wxz, Vxz, Wxh, Vxh, bxz, bhz, bxh, bhh)
